# Initial kernel scaffold; baseline (speedup 1.0000x reference)
import jax
import jax.numpy as jnp
from jax.experimental import pallas as pl


def kernel(x, edge_index, edge_attr, sequence_data, peptide_property, params):
    raise NotImplementedError("write your pallas kernel here")


# speedup vs baseline: 2.4526x; 2.4526x over previous
import functools

import jax
import jax.numpy as jnp
import numpy as np
from jax import lax
from jax.experimental import pallas as pl
from jax.experimental.pallas import tpu as pltpu
from jax.experimental.pallas import tpu_sc as plsc

NN = 10000
NE = 320000
NG = 20
NPG = 500
SP = 512
HID = 64
NH = 8
DH = HID // NH
W = 80
CO = 64

NC, NS = 2, 16
NWK = NC * NS
EPW = NE // NWK
CH = 80
NCHUNK = EPW // CH

BE = 2000
BN = 2000

def _silu(x):
    return x / (1.0 + jnp.exp(-x))


def _bdot(a, b):
    return jnp.dot(a.astype(jnp.bfloat16), b.astype(jnp.bfloat16),
                   preferred_element_type=jnp.float32)


def _b(x):
    return x.astype(jnp.bfloat16).astype(jnp.float32)


@functools.cache
def _build_sc_kernels():
    mesh = plsc.VectorSubcoreMesh(core_axis_name="c", subcore_axis_name="s",
                                  num_cores=NC, num_subcores=NS)

    @functools.partial(
        pl.kernel,
        out_type=(
            jax.ShapeDtypeStruct((NE, W), jnp.float32),
            jax.ShapeDtypeStruct((NE, W), jnp.float32),
        ),
        mesh=mesh,
        scratch_types=[
            pltpu.VMEM((NCHUNK, CH), jnp.int32),
            pltpu.VMEM((NCHUNK, CH), jnp.int32),
            pltpu.VMEM((CH, W), jnp.float32),
            pltpu.SemaphoreType.DMA,
        ],
        compiler_params=pltpu.CompilerParams(use_tc_tiling_on_sc=False),
    )
    def _sc_gather(tab, src3d, dst3d, fs_out, fd_out, idx_s, idx_d, rows, sem):
        c = lax.axis_index("c")
        s = lax.axis_index("s")
        wid = s * NC + c
        base = wid * NCHUNK
        pltpu.sync_copy(src3d.at[wid], idx_s)
        pltpu.sync_copy(dst3d.at[wid], idx_d)

        def body(j, carry):
            off = (base + j) * CH
            pltpu.async_copy(tab.at[idx_s.at[j]], rows, sem).wait()
            pltpu.sync_copy(rows, fs_out.at[pl.ds(off, CH)])
            pltpu.async_copy(tab.at[idx_d.at[j]], rows, sem).wait()
            pltpu.sync_copy(rows, fd_out.at[pl.ds(off, CH)])
            return carry

        lax.fori_loop(0, NCHUNK, body, 0)

    @functools.partial(
        pl.kernel,
        out_type=jax.ShapeDtypeStruct((NC, NN, W), jnp.float32),
        mesh=mesh,
        scratch_types=[
            pltpu.VMEM((NCHUNK, CH), jnp.int32),
            pltpu.VMEM((CH, W), jnp.float32),
            pltpu.VMEM_SHARED((NN, W), jnp.float32),
        ],
        compiler_params=pltpu.CompilerParams(use_tc_tiling_on_sc=False),
    )
    def _sc_scatter(msg, dst3d, zeros_hbm, out, idx_d, msg_v, accum):
        c = lax.axis_index("c")
        s = lax.axis_index("s")
        wid = s * NC + c
        base = wid * NCHUNK
        rpt = 624
        pltpu.sync_copy(zeros_hbm.at[pl.ds(s * rpt, rpt)],
                        accum.at[pl.ds(s * rpt, rpt)])

        @pl.when(s == NS - 1)
        def _():
            pltpu.sync_copy(zeros_hbm.at[pl.ds(NS * rpt, NN - NS * rpt)],
                            accum.at[pl.ds(NS * rpt, NN - NS * rpt)])

        pltpu.sync_copy(dst3d.at[wid], idx_d)
        plsc.subcore_barrier()

        def body(j, carry):
            off = (base + j) * CH
            pltpu.sync_copy(msg.at[pl.ds(off, CH)], msg_v)
            pltpu.sync_copy(msg_v, accum.at[idx_d.at[j]], add=True)
            return carry

        lax.fori_loop(0, NCHUNK, body, 0)
        plsc.subcore_barrier()
        pltpu.sync_copy(accum.at[pl.ds(s * rpt, rpt)],
                        out.at[c, pl.ds(s * rpt, rpt)])

        @pl.when(s == NS - 1)
        def _():
            pltpu.sync_copy(accum.at[pl.ds(NS * rpt, NN - NS * rpt)],
                            out.at[c, pl.ds(NS * rpt, NN - NS * rpt)])

    return _sc_gather, _sc_scatter


def _edge_body(ind, fs_ref, fd_ref, ea_ref, ew1_ref, eb1_ref,
               ew2_ref, eb2_ref, cw1_ref, cb1_ref, cw2_ref, out_ref):
    fs = fs_ref[...]
    fd = fd_ref[...]
    xd = fs[:, CO:W] - fd[:, CO:W]
    radial = (xd[:, 0:1] * xd[:, 0:1] + xd[:, 1:2] * xd[:, 1:2]
              + xd[:, 2:3] * xd[:, 2:3])
    xdn = xd / (jnp.sqrt(radial) + 1e-30)
    ew1 = ew1_ref[...]
    if ind == 20:
        f = jnp.concatenate([fs[:, 0:ind], fd[:, 0:ind], radial, ea_ref[...]],
                            axis=1)
        m1 = _bdot(f, ew1) + eb1_ref[...]
    else:
        f = jnp.concatenate([fs[:, 0:ind], fd[:, 0:ind]], axis=1)
        m1 = _bdot(f, ew1[0:2 * ind]) \
            + radial * ew1[2 * ind:2 * ind + 1] \
            + ea_ref[...] * ew1[2 * ind + 1:2 * ind + 2] + eb1_ref[...]
    m1 = _silu(m1)
    mh = _silu(_bdot(m1, ew2_ref[...]) + eb2_ref[...])
    cw = _bdot(_silu(_bdot(mh, cw1_ref[...]) + cb1_ref[...]),
               cw2_ref[...])
    mx = cw * xdn
    lane = lax.broadcasted_iota(jnp.int32, mx.shape, 1)
    mx = jnp.where(lane == 3, jnp.float32(1.0), jnp.where(lane < 3, mx, 0.0))
    out_ref[...] = jnp.concatenate([mh, mx], axis=1)


def _edge_call(ind, fs, fd, ea, ew1, eb1, ew2, eb2, cw1, cb1, cw2):
    nblk = NE // BE
    full = lambda shape: pl.BlockSpec(shape, lambda i: (0, 0))
    return pl.pallas_call(
        functools.partial(_edge_body, ind),
        grid=(nblk,),
        in_specs=[
            pl.BlockSpec((BE, W), lambda i: (i, 0)),
            pl.BlockSpec((BE, W), lambda i: (i, 0)),
            pl.BlockSpec((BE, 1), lambda i: (i, 0)),
            full((2 * ind + 2, HID)), full((1, HID)),
            full((HID, HID)), full((1, HID)), full((HID, HID)), full((1, HID)),
            full((HID, 1)),
        ],
        out_specs=pl.BlockSpec((BE, W), lambda i: (i, 0)),
        out_shape=jax.ShapeDtypeStruct((NE, W), jnp.float32),
    )(fs, fd, ea, ew1, eb1, ew2, eb2, cw1, cb1, cw2)


def _node_body(ind, tab_ref, p_ref, nw1_ref, nb1_ref, nw2_ref, nb2_ref,
               out_ref):
    tab = tab_ref[...]
    acc = p_ref[0] + p_ref[1]
    h = tab[:, 0:ind]
    hn = acc[:, 0:CO]
    deg = jnp.maximum(acc[:, CO + 3:CO + 4], 1.0)
    dx = acc[:, CO:W] / deg
    lane = lax.broadcasted_iota(jnp.int32, dx.shape, 1)
    dx = jnp.where(lane < 3, dx, 0.0)
    coord_new = tab[:, CO:W] + dx
    nw1 = nw1_ref[...]
    hi = _silu(_bdot(h, nw1[0:ind]) + _bdot(hn, nw1[ind:]) + nb1_ref[...])
    h_new = _bdot(hi, nw2_ref[...]) + nb2_ref[...]
    out_ref[...] = jnp.concatenate([h_new, coord_new], axis=1)


def _node_call(ind, tab, partials, nw1, nb1, nw2, nb2):
    nblk = NN // BN
    full = lambda shape: pl.BlockSpec(shape, lambda i: (0, 0))
    return pl.pallas_call(
        functools.partial(_node_body, ind),
        grid=(nblk,),
        in_specs=[
            pl.BlockSpec((BN, W), lambda i: (i, 0)),
            pl.BlockSpec((NC, BN, W), lambda i: (0, i, 0)),
            full((ind + HID, HID)), full((1, HID)),
            full((HID, HID)), full((1, HID)),
        ],
        out_specs=pl.BlockSpec((BN, W), lambda i: (i, 0)),
        out_shape=jax.ShapeDtypeStruct((NN, W), jnp.float32),
    )(tab, partials, nw1, nb1, nw2, nb2)


def _attn_body(h_ref, wq_ref, bq_ref, wk_ref, bk_ref, wv_ref, bv_ref,
               wo_ref, bo_ref, clsw_ref, clsb_ref, hw_ref, hb_ref,
               nw_ref, nb_ref, fo_ref, np_ref):
    hb = h_ref[0]
    q = _bdot(hb, wq_ref[...]) + bq_ref[...]
    k = _bdot(hb, wk_ref[...]) + bk_ref[...]
    v = _bdot(hb, wv_ref[...]) + bv_ref[...]
    scale = jnp.float32(1.0 / np.sqrt(DH))
    kmask = lax.broadcasted_iota(jnp.int32, (SP, SP), 1) < NPG
    outs = []
    for hd in range(NH):
        sl = slice(hd * DH, (hd + 1) * DH)
        qh, kh, vh = q[:, sl], k[:, sl], v[:, sl]
        lg = lax.dot_general(qh.astype(jnp.bfloat16), kh.astype(jnp.bfloat16),
                             (((1,), (1,)), ((), ())),
                             preferred_element_type=jnp.float32) * scale
        lg = jnp.where(kmask, lg, jnp.float32(-1e30))
        m = jnp.max(lg, axis=1, keepdims=True)
        e = jnp.exp(lg - m)
        wsm = e / jnp.sum(e, axis=1, keepdims=True)
        outs.append(_bdot(wsm, vh))
    attn = jnp.concatenate(outs, axis=1)
    attn = _bdot(attn, wo_ref[...]) + bo_ref[...]
    qmask = lax.broadcasted_iota(jnp.int32, (SP, HID), 0) < NPG
    attn = jnp.where(qmask, attn, 0.0)
    pooled = jnp.sum(attn, axis=0, keepdims=True) * jnp.float32(1.0 / NPG)
    fus = jnp.maximum(_bdot(pooled, clsw_ref[...]) + clsb_ref[...], 0.0)
    i = pl.program_id(0)
    fo_ref[pl.ds(i, 1), :] = _bdot(fus, hw_ref[...]) + hb_ref[...]
    np_ref[pl.ds(i, 1), :] = _bdot(fus, nw_ref[...]) + nb_ref[...]


def _attn_call(h3, p):
    full = lambda shape: pl.BlockSpec(shape, lambda i: (0, 0))
    return pl.pallas_call(
        _attn_body,
        grid=(NG,),
        in_specs=[
            pl.BlockSpec((1, SP, HID), lambda i: (i, 0, 0)),
            full((HID, HID)), full((1, HID)),
            full((HID, HID)), full((1, HID)),
            full((HID, HID)), full((1, HID)),
            full((HID, HID)), full((1, HID)),
            full((HID, 32)), full((1, 32)),
            full((32, 1)), full((1, 1)),
            full((32, 20)), full((1, 20)),
        ],
        out_specs=[
            pl.BlockSpec((NG, 1), lambda i: (0, 0)),
            pl.BlockSpec((NG, 20), lambda i: (0, 0)),
        ],
        out_shape=[
            jax.ShapeDtypeStruct((NG, 1), jnp.float32),
            jax.ShapeDtypeStruct((NG, 20), jnp.float32),
        ],
    )(h3, p["Wq"], p["bq"].reshape(1, HID), p["Wk"], p["bk"].reshape(1, HID),
      p["Wv"], p["bv"].reshape(1, HID), p["Wo"], p["bo"].reshape(1, HID),
      p["cls_W"], p["cls_b"].reshape(1, 32), p["head_W"],
      p["head_b"].reshape(1, 1), p["node_W"], p["node_b"].reshape(1, 20))


def kernel(x, edge_index, edge_attr, sequence_data, peptide_property, params):
    f32 = jnp.float32
    src3d = edge_index[0].reshape(NWK, NCHUNK, CH)
    dst3d = edge_index[1].reshape(NWK, NCHUNK, CH)
    ea = edge_attr.astype(f32)

    tab = jnp.concatenate(
        [x[:, :20], jnp.zeros((NN, CO - 20), f32),
         x[:, 20:23], jnp.zeros((NN, W - CO - 3), f32)], axis=1)
    zeros_acc = jnp.zeros((NN, W), f32)
    sc_gather, sc_scatter = _build_sc_kernels()

    for li, p in enumerate(params["layers"]):
        in_dim = 20 if li == 0 else HID
        fs, fd = sc_gather(tab, src3d, dst3d)
        msg = _edge_call(in_dim, fs, fd, ea, p["eW1"],
                         p["eb1"].reshape(1, HID), p["eW2"],
                         p["eb2"].reshape(1, HID), p["cW1"],
                         p["cb1"].reshape(1, HID), p["cW2"])
        partials = sc_scatter(msg, dst3d, zeros_acc)
        tab = _node_call(in_dim, tab, partials, p["nW1"],
                         p["nb1"].reshape(1, HID), p["nW2"],
                         p["nb2"].reshape(1, HID))

    h3 = tab[:, :HID].reshape(NG, NPG, HID)
    h3 = jnp.pad(h3, ((0, 0), (0, SP - NPG), (0, 0)))
    final_output, node_prediction = _attn_call(h3, params)
    return final_output, node_prediction


# speedup vs baseline: 2.5386x; 1.0351x over previous
import functools

import jax
import jax.numpy as jnp
import numpy as np
from jax import lax
from jax.experimental import pallas as pl
from jax.experimental.pallas import tpu as pltpu
from jax.experimental.pallas import tpu_sc as plsc

NN = 10000
NE = 320000
NG = 20
NPG = 500
SP = 512
HID = 64
NH = 8
DH = HID // NH
W = 80
CO = 64

NC, NS = 2, 16
NWK = NC * NS
EPW = NE // NWK
CH = 80
NCHUNK = EPW // CH

BE = 2000
BN = 2000

def _silu(x):
    return x / (1.0 + jnp.exp(-x))


def _bdot(a, b):
    return jnp.dot(a.astype(jnp.bfloat16), b.astype(jnp.bfloat16),
                   preferred_element_type=jnp.float32)


def _b(x):
    return x.astype(jnp.bfloat16).astype(jnp.float32)


@functools.cache
def _build_sc_kernels():
    mesh = plsc.VectorSubcoreMesh(core_axis_name="c", subcore_axis_name="s",
                                  num_cores=NC, num_subcores=NS)

    @functools.partial(
        pl.kernel,
        out_type=(
            jax.ShapeDtypeStruct((NE, W), jnp.float32),
            jax.ShapeDtypeStruct((NE, W), jnp.float32),
        ),
        mesh=mesh,
        scratch_types=[
            pltpu.VMEM((NCHUNK, CH), jnp.int32),
            pltpu.VMEM((NCHUNK, CH), jnp.int32),
            pltpu.VMEM((CH, W), jnp.float32),
            pltpu.SemaphoreType.DMA,
        ],
        compiler_params=pltpu.CompilerParams(use_tc_tiling_on_sc=False),
    )
    def _sc_gather(tab, src3d, dst3d, fs_out, fd_out, idx_s, idx_d, rows, sem):
        c = lax.axis_index("c")
        s = lax.axis_index("s")
        wid = s * NC + c
        base = wid * NCHUNK
        pltpu.sync_copy(src3d.at[wid], idx_s)
        pltpu.sync_copy(dst3d.at[wid], idx_d)

        def body(j, carry):
            off = (base + j) * CH
            pltpu.async_copy(tab.at[idx_s.at[j]], rows, sem).wait()
            pltpu.sync_copy(rows, fs_out.at[pl.ds(off, CH)])
            pltpu.async_copy(tab.at[idx_d.at[j]], rows, sem).wait()
            pltpu.sync_copy(rows, fd_out.at[pl.ds(off, CH)])
            return carry

        lax.fori_loop(0, NCHUNK, body, 0)

    @functools.partial(
        pl.kernel,
        out_type=jax.ShapeDtypeStruct((NC, NN, W), jnp.float32),
        mesh=mesh,
        scratch_types=[
            pltpu.VMEM((NCHUNK, CH), jnp.int32),
            pltpu.VMEM((CH, W), jnp.float32),
            pltpu.VMEM_SHARED((NN, W), jnp.float32),
        ],
        compiler_params=pltpu.CompilerParams(use_tc_tiling_on_sc=False),
    )
    def _sc_scatter(msg, dst3d, zeros_hbm, out, idx_d, msg_v, accum):
        c = lax.axis_index("c")
        s = lax.axis_index("s")
        wid = s * NC + c
        base = wid * NCHUNK
        rpt = 624
        pltpu.sync_copy(zeros_hbm.at[pl.ds(s * rpt, rpt)],
                        accum.at[pl.ds(s * rpt, rpt)])

        @pl.when(s == NS - 1)
        def _():
            pltpu.sync_copy(zeros_hbm.at[pl.ds(NS * rpt, NN - NS * rpt)],
                            accum.at[pl.ds(NS * rpt, NN - NS * rpt)])

        pltpu.sync_copy(dst3d.at[wid], idx_d)
        plsc.subcore_barrier()

        def body(j, carry):
            off = (base + j) * CH
            pltpu.sync_copy(msg.at[pl.ds(off, CH)], msg_v)
            pltpu.sync_copy(msg_v, accum.at[idx_d.at[j]], add=True)
            return carry

        lax.fori_loop(0, NCHUNK, body, 0)
        plsc.subcore_barrier()
        pltpu.sync_copy(accum.at[pl.ds(s * rpt, rpt)],
                        out.at[c, pl.ds(s * rpt, rpt)])

        @pl.when(s == NS - 1)
        def _():
            pltpu.sync_copy(accum.at[pl.ds(NS * rpt, NN - NS * rpt)],
                            out.at[c, pl.ds(NS * rpt, NN - NS * rpt)])

    return _sc_gather, _sc_scatter


def _edge_body(ind, fs_ref, fd_ref, ea_ref, ew1_ref, eb1_ref,
               ew2_ref, eb2_ref, cw1_ref, cb1_ref, cw2_ref, out_ref):
    fs = fs_ref[...]
    fd = fd_ref[...]
    xd = fs[:, CO:W] - fd[:, CO:W]
    radial = (xd[:, 0:1] * xd[:, 0:1] + xd[:, 1:2] * xd[:, 1:2]
              + xd[:, 2:3] * xd[:, 2:3])
    xdn = xd / (jnp.sqrt(radial) + 1e-30)
    ew1 = ew1_ref[...]
    if ind == 20:
        f = jnp.concatenate([fs[:, 0:ind], fd[:, 0:ind], radial, ea_ref[...]],
                            axis=1)
        m1 = _bdot(f, ew1) + eb1_ref[...]
    else:
        f = jnp.concatenate([fs[:, 0:ind], fd[:, 0:ind], radial, ea_ref[...]],
                            axis=1)
        m1 = _bdot(f, ew1) + eb1_ref[...]
    m1 = _silu(m1)
    mh = _silu(_bdot(m1, ew2_ref[...]) + eb2_ref[...])
    cw = _bdot(_silu(_bdot(mh, cw1_ref[...]) + cb1_ref[...]),
               cw2_ref[...])
    mx = cw * xdn
    lane = lax.broadcasted_iota(jnp.int32, mx.shape, 1)
    mx = jnp.where(lane == 3, jnp.float32(1.0), jnp.where(lane < 3, mx, 0.0))
    out_ref[...] = jnp.concatenate([mh, mx], axis=1)


def _edge_call(ind, fs, fd, ea, ew1, eb1, ew2, eb2, cw1, cb1, cw2):
    nblk = NE // BE
    full = lambda shape: pl.BlockSpec(shape, lambda i: (0, 0))
    return pl.pallas_call(
        functools.partial(_edge_body, ind),
        grid=(nblk,),
        in_specs=[
            pl.BlockSpec((BE, W), lambda i: (i, 0)),
            pl.BlockSpec((BE, W), lambda i: (i, 0)),
            pl.BlockSpec((BE, 1), lambda i: (i, 0)),
            full((2 * ind + 2, HID)), full((1, HID)),
            full((HID, HID)), full((1, HID)), full((HID, HID)), full((1, HID)),
            full((HID, 1)),
        ],
        out_specs=pl.BlockSpec((BE, W), lambda i: (i, 0)),
        out_shape=jax.ShapeDtypeStruct((NE, W), jnp.float32),
    )(fs, fd, ea, ew1, eb1, ew2, eb2, cw1, cb1, cw2)


def _node_body(ind, tab_ref, p_ref, nw1_ref, nb1_ref, nw2_ref, nb2_ref,
               out_ref):
    tab = tab_ref[...]
    acc = p_ref[0] + p_ref[1]
    h = tab[:, 0:ind]
    hn = acc[:, 0:CO]
    deg = jnp.maximum(acc[:, CO + 3:CO + 4], 1.0)
    dx = acc[:, CO:W] / deg
    lane = lax.broadcasted_iota(jnp.int32, dx.shape, 1)
    dx = jnp.where(lane < 3, dx, 0.0)
    coord_new = tab[:, CO:W] + dx
    h_in = jnp.concatenate([h, hn], axis=1)
    hi = _silu(_bdot(h_in, nw1_ref[...]) + nb1_ref[...])
    h_new = _bdot(hi, nw2_ref[...]) + nb2_ref[...]
    out_ref[...] = jnp.concatenate([h_new, coord_new], axis=1)


def _node_call(ind, tab, partials, nw1, nb1, nw2, nb2):
    nblk = NN // BN
    full = lambda shape: pl.BlockSpec(shape, lambda i: (0, 0))
    return pl.pallas_call(
        functools.partial(_node_body, ind),
        grid=(nblk,),
        in_specs=[
            pl.BlockSpec((BN, W), lambda i: (i, 0)),
            pl.BlockSpec((NC, BN, W), lambda i: (0, i, 0)),
            full((ind + HID, HID)), full((1, HID)),
            full((HID, HID)), full((1, HID)),
        ],
        out_specs=pl.BlockSpec((BN, W), lambda i: (i, 0)),
        out_shape=jax.ShapeDtypeStruct((NN, W), jnp.float32),
    )(tab, partials, nw1, nb1, nw2, nb2)


def _attn_body(h_ref, wq_ref, bq_ref, wk_ref, bk_ref, wv_ref, bv_ref,
               wo_ref, bo_ref, clsw_ref, clsb_ref, hw_ref, hb_ref,
               nw_ref, nb_ref, fo_ref, np_ref):
    hb = h_ref[0]
    q = _bdot(hb, wq_ref[...]) + bq_ref[...]
    k = _bdot(hb, wk_ref[...]) + bk_ref[...]
    v = _bdot(hb, wv_ref[...]) + bv_ref[...]
    scale = jnp.float32(1.0 / np.sqrt(DH))
    kmask = lax.broadcasted_iota(jnp.int32, (SP, SP), 1) < NPG
    outs = []
    for hd in range(NH):
        sl = slice(hd * DH, (hd + 1) * DH)
        qh, kh, vh = q[:, sl], k[:, sl], v[:, sl]
        lg = lax.dot_general(qh.astype(jnp.bfloat16), kh.astype(jnp.bfloat16),
                             (((1,), (1,)), ((), ())),
                             preferred_element_type=jnp.float32) * scale
        lg = jnp.where(kmask, lg, jnp.float32(-1e30))
        m = jnp.max(lg, axis=1, keepdims=True)
        e = jnp.exp(lg - m)
        wsm = e / jnp.sum(e, axis=1, keepdims=True)
        outs.append(_bdot(wsm, vh))
    attn = jnp.concatenate(outs, axis=1)
    attn = _bdot(attn, wo_ref[...]) + bo_ref[...]
    qmask = lax.broadcasted_iota(jnp.int32, (SP, HID), 0) < NPG
    attn = jnp.where(qmask, attn, 0.0)
    pooled = jnp.sum(attn, axis=0, keepdims=True) * jnp.float32(1.0 / NPG)
    fus = jnp.maximum(_bdot(pooled, clsw_ref[...]) + clsb_ref[...], 0.0)
    i = pl.program_id(0)
    fo_ref[pl.ds(i, 1), :] = _bdot(fus, hw_ref[...]) + hb_ref[...]
    np_ref[pl.ds(i, 1), :] = _bdot(fus, nw_ref[...]) + nb_ref[...]


def _attn_call(h3, p):
    full = lambda shape: pl.BlockSpec(shape, lambda i: (0, 0))
    return pl.pallas_call(
        _attn_body,
        grid=(NG,),
        in_specs=[
            pl.BlockSpec((1, SP, HID), lambda i: (i, 0, 0)),
            full((HID, HID)), full((1, HID)),
            full((HID, HID)), full((1, HID)),
            full((HID, HID)), full((1, HID)),
            full((HID, HID)), full((1, HID)),
            full((HID, 32)), full((1, 32)),
            full((32, 1)), full((1, 1)),
            full((32, 20)), full((1, 20)),
        ],
        out_specs=[
            pl.BlockSpec((NG, 1), lambda i: (0, 0)),
            pl.BlockSpec((NG, 20), lambda i: (0, 0)),
        ],
        out_shape=[
            jax.ShapeDtypeStruct((NG, 1), jnp.float32),
            jax.ShapeDtypeStruct((NG, 20), jnp.float32),
        ],
    )(h3, p["Wq"], p["bq"].reshape(1, HID), p["Wk"], p["bk"].reshape(1, HID),
      p["Wv"], p["bv"].reshape(1, HID), p["Wo"], p["bo"].reshape(1, HID),
      p["cls_W"], p["cls_b"].reshape(1, 32), p["head_W"],
      p["head_b"].reshape(1, 1), p["node_W"], p["node_b"].reshape(1, 20))


def kernel(x, edge_index, edge_attr, sequence_data, peptide_property, params):
    f32 = jnp.float32
    src3d = edge_index[0].reshape(NWK, NCHUNK, CH)
    dst3d = edge_index[1].reshape(NWK, NCHUNK, CH)
    ea = edge_attr.astype(f32)

    tab = jnp.concatenate(
        [x[:, :20], jnp.zeros((NN, CO - 20), f32),
         x[:, 20:23], jnp.zeros((NN, W - CO - 3), f32)], axis=1)
    zeros_acc = jnp.zeros((NN, W), f32)
    sc_gather, sc_scatter = _build_sc_kernels()

    for li, p in enumerate(params["layers"]):
        in_dim = 20 if li == 0 else HID
        fs, fd = sc_gather(tab, src3d, dst3d)
        msg = _edge_call(in_dim, fs, fd, ea, p["eW1"],
                         p["eb1"].reshape(1, HID), p["eW2"],
                         p["eb2"].reshape(1, HID), p["cW1"],
                         p["cb1"].reshape(1, HID), p["cW2"])
        partials = sc_scatter(msg, dst3d, zeros_acc)
        tab = _node_call(in_dim, tab, partials, p["nW1"],
                         p["nb1"].reshape(1, HID), p["nW2"],
                         p["nb2"].reshape(1, HID))

    h3 = tab[:, :HID].reshape(NG, NPG, HID)
    h3 = jnp.pad(h3, ((0, 0), (0, SP - NPG), (0, 0)))
    final_output, node_prediction = _attn_call(h3, params)
    return final_output, node_prediction
